# Initial kernel scaffold; baseline (speedup 1.0000x reference)
import jax
import jax.numpy as jnp
from jax.experimental import pallas as pl


def kernel(prop_state, A, W_in, b_in, W_out, b_out, W_r, b_r, W_z, b_z, W_t, b_t):
    raise NotImplementedError("write your pallas kernel here")


# speedup vs baseline: 1.2197x; 1.2197x over previous
import jax
import jax.numpy as jnp
from jax.experimental import pallas as pl

_N_STEPS = 5
_N = 4096
_D = 256
_RB = 256
_NRB = _N // _RB
_BF = jnp.bfloat16
_F32 = jnp.float32


def _prologue_body(p_ref, w_in_t_ref, w_out_t_ref, b_in_ref, b_out_ref,
                   s_in_ref, s_out_ref):
    pb = p_ref[...].astype(_BF)
    s_in = jnp.dot(pb, w_in_t_ref[...], preferred_element_type=_F32) + b_in_ref[...]
    s_out = jnp.dot(pb, w_out_t_ref[...], preferred_element_type=_F32) + b_out_ref[...]
    s_in_ref[...] = s_in.astype(_BF)
    s_out_ref[...] = s_out.astype(_BF)


def _step_body(a_ref, s_in_ref, s_out_ref, p_ref,
               wr_ref, wz_ref, wt_ref, w_in_t_ref, w_out_t_ref,
               b_r_ref, b_z_ref, b_t_ref, b_in_ref, b_out_ref,
               p_new_ref, s_in_next_ref, s_out_next_ref):
    ai = jnp.dot(a_ref[0], s_in_ref[...], preferred_element_type=_F32)
    ao = jnp.dot(a_ref[1], s_out_ref[...], preferred_element_type=_F32)
    p = p_ref[...]

    a_full = jnp.concatenate([ai.astype(_BF), ao.astype(_BF), p.astype(_BF)],
                             axis=1)
    wr = wr_ref[...]
    wz = wz_ref[...]
    wt = wt_ref[...]

    pre_r = jnp.dot(a_full, wr, preferred_element_type=_F32) + b_r_ref[...]
    pre_z = jnp.dot(a_full, wz, preferred_element_type=_F32) + b_z_ref[...]
    r = jax.nn.sigmoid(pre_r)
    z = jax.nn.sigmoid(pre_z)

    joined = jnp.concatenate([a_full[:, : 2 * _D], (r * p).astype(_BF)], axis=1)
    pre_h = jnp.dot(joined, wt, preferred_element_type=_F32) + b_t_ref[...]
    h = jnp.tanh(pre_h)

    p_new = (1.0 - z) * p + z * h
    p_new_ref[...] = p_new

    pnb = p_new.astype(_BF)
    s_in_next = jnp.dot(pnb, w_in_t_ref[...], preferred_element_type=_F32) + b_in_ref[...]
    s_out_next = jnp.dot(pnb, w_out_t_ref[...], preferred_element_type=_F32) + b_out_ref[...]
    s_in_next_ref[...] = s_in_next.astype(_BF)
    s_out_next_ref[...] = s_out_next.astype(_BF)


def _full(shape):
    return pl.BlockSpec(shape, lambda i: (0,) * len(shape))


def _make_calls(interpret=False):
    prologue = pl.pallas_call(
        _prologue_body,
        grid=(4,),
        in_specs=[
            pl.BlockSpec((_N // 4, _D), lambda i: (i, 0)),
            _full((_D, _D)),
            _full((_D, _D)),
            _full((1, _D)),
            _full((1, _D)),
        ],
        out_specs=[
            pl.BlockSpec((_N // 4, _D), lambda i: (i, 0)),
            pl.BlockSpec((_N // 4, _D), lambda i: (i, 0)),
        ],
        out_shape=[
            jax.ShapeDtypeStruct((_N, _D), _BF),
            jax.ShapeDtypeStruct((_N, _D), _BF),
        ],
        interpret=interpret,
    )
    step = pl.pallas_call(
        _step_body,
        grid=(_NRB,),
        in_specs=[
            pl.BlockSpec((2, _RB, _N), lambda i: (0, i, 0)),
            _full((_N, _D)),
            _full((_N, _D)),
            pl.BlockSpec((_RB, _D), lambda i: (i, 0)),
            _full((3 * _D, _D)),
            _full((3 * _D, _D)),
            _full((3 * _D, _D)),
            _full((_D, _D)),
            _full((_D, _D)),
            _full((1, _D)),
            _full((1, _D)),
            _full((1, _D)),
            _full((1, _D)),
            _full((1, _D)),
        ],
        out_specs=[
            pl.BlockSpec((_RB, _D), lambda i: (i, 0)),
            pl.BlockSpec((_RB, _D), lambda i: (i, 0)),
            pl.BlockSpec((_RB, _D), lambda i: (i, 0)),
        ],
        out_shape=[
            jax.ShapeDtypeStruct((_N, _D), _F32),
            jax.ShapeDtypeStruct((_N, _D), _BF),
            jax.ShapeDtypeStruct((_N, _D), _BF),
        ],
        interpret=interpret,
    )
    return prologue, step


_PROLOGUE, _STEP = _make_calls()


def _kernel_impl(prologue, step, prop_state, A, W_in, b_in, W_out, b_out,
                 W_r, b_r, W_z, b_z, W_t, b_t):
    a_bf = A.astype(_BF)
    w_in_t = W_in.T.astype(_BF)
    w_out_t = W_out.T.astype(_BF)
    wr_t = W_r.T.astype(_BF)
    wz_t = W_z.T.astype(_BF)
    wt_t = W_t.T.astype(_BF)
    b_in2 = b_in.reshape(1, _D)
    b_out2 = b_out.reshape(1, _D)
    b_r2 = b_r.reshape(1, _D)
    b_z2 = b_z.reshape(1, _D)
    b_t2 = b_t.reshape(1, _D)

    s_in, s_out = prologue(prop_state, w_in_t, w_out_t, b_in2, b_out2)
    p = prop_state
    for _ in range(_N_STEPS):
        p, s_in, s_out = step(a_bf, s_in, s_out, p, wr_t, wz_t, wt_t,
                              w_in_t, w_out_t, b_r2, b_z2, b_t2, b_in2, b_out2)
    return p


def kernel(prop_state, A, W_in, b_in, W_out, b_out, W_r, b_r, W_z, b_z, W_t, b_t):
    return _kernel_impl(_PROLOGUE, _STEP, prop_state, A, W_in, b_in, W_out,
                        b_out, W_r, b_r, W_z, b_z, W_t, b_t)
